# baseline (speedup 1.0000x reference)
import functools

import jax
import jax.numpy as jnp
from jax import lax
from jax.experimental import pallas as pl
from jax.experimental.pallas import tpu as pltpu
from jax.experimental.pallas import tpu_sc as plsc

_NUM_CORES = 2
_NUM_SUBCORES = 16
_NUM_WORKERS = _NUM_CORES * _NUM_SUBCORES
_BATCH = 256
_BPW = _BATCH // _NUM_WORKERS
_NQ = 1024
_LANES = 16
_WORDS = _NQ // 4
_STEPS = _WORDS // _LANES


def _ellipse_body(boxes_hbm, out_hbm, box_v, out_v):
  wid = lax.axis_index("s") * _NUM_CORES + lax.axis_index("c")
  pltpu.sync_copy(
      boxes_hbm.at[pl.ds(wid * (4 * _BPW), 4 * _BPW)],
      box_v.at[pl.ds(0, 4 * _BPW)],
  )
  lane = lax.iota(jnp.int32, _LANES)
  lane4 = lane * 4
  pxs = [((lane4 + k) & 31).astype(jnp.float32) for k in range(4)]
  py_half = (lane >> 3).astype(jnp.float32)
  weights = [jnp.int32(1 << (8 * k)) for k in range(4)]
  zero = jnp.zeros((_LANES,), jnp.int32)
  one = jnp.ones((_LANES,), jnp.float32)
  for i in range(_BPW):
    bvec = box_v[pl.ds(4 * i, _LANES)]
    tcx = jnp.broadcast_to(bvec[0], (_LANES,)) * 32.0
    tcy = jnp.broadcast_to(bvec[1], (_LANES,)) * 32.0
    tw = jnp.broadcast_to(bvec[2], (_LANES,)) * 32.0
    th = jnp.broadcast_to(bvec[3], (_LANES,)) * 32.0
    ra = tw / 2.0 + 1e-06
    rb = th / 2.0 + 1e-06
    ia = one / (ra * ra)
    ib = one / (rb * rb)
    qxs = []
    for k in range(4):
      dx = tcx - pxs[k]
      qxs.append(dx * dx * ia)
    cyb = tcy - py_half

    for j in range(_STEPS):
      dy = cyb - float(2 * j)
      t = dy * dy * ib
      acc = jnp.where(qxs[0] + t < 1.0, weights[0], zero)
      for k in range(1, 4):
        acc = acc + jnp.where(qxs[k] + t < 1.0, weights[k], zero)
      out_v[i, pl.ds(j * _LANES, _LANES)] = acc
  pltpu.sync_copy(out_v, out_hbm.at[pl.ds(wid * _BPW, _BPW)])


@jax.jit
def _ellipse_mask(boxes_flat):
  mesh = plsc.VectorSubcoreMesh(core_axis_name="c", subcore_axis_name="s")
  f = functools.partial(
      pl.kernel,
      mesh=mesh,
      out_type=jax.ShapeDtypeStruct((_BATCH, _WORDS), jnp.int32),
      scratch_types=[
          pltpu.VMEM((4 * _BPW + _LANES,), jnp.float32),
          pltpu.VMEM((_BPW, _WORDS), jnp.int32),
      ],
  )(_ellipse_body)
  return f(boxes_flat)


def kernel(pred_logits, boxes):
  del pred_logits
  words = _ellipse_mask(boxes.reshape(_BATCH * 4))
  mask_bytes = lax.bitcast_convert_type(words, jnp.uint8).reshape(_BATCH, _NQ)
  return mask_bytes.astype(jnp.bool_)


# speedup vs baseline: 1.0507x; 1.0507x over previous
import functools

import jax
import jax.numpy as jnp
from jax import lax
from jax.experimental import pallas as pl
from jax.experimental.pallas import tpu as pltpu
from jax.experimental.pallas import tpu_sc as plsc

_NUM_CORES = 2
_NUM_SUBCORES = 16
_NUM_WORKERS = _NUM_CORES * _NUM_SUBCORES
_BATCH = 256
_BPW = _BATCH // _NUM_WORKERS
_NQ = 1024
_LANES = 16
_WORDS = _NQ // 4
_STEPS = _WORDS // _LANES


def _ellipse_body(boxes_hbm, out_hbm, box_v, out_v):
  wid = lax.axis_index("s") * _NUM_CORES + lax.axis_index("c")
  pltpu.sync_copy(
      boxes_hbm.at[pl.ds(wid * (4 * _BPW), 4 * _BPW)],
      box_v.at[pl.ds(0, 4 * _BPW)],
  )
  lane = lax.iota(jnp.int32, _LANES)
  lane4 = lane * 4
  pxs = [((lane4 + k) & 31).astype(jnp.float32) for k in range(4)]
  py_half = (lane >> 3).astype(jnp.float32)
  weights = [jnp.int32(1 << (8 * k)) for k in range(4)]
  zero = jnp.zeros((_LANES,), jnp.int32)
  one = jnp.ones((_LANES,), jnp.float32)
  def batch_body(i, carry):
    bvec = box_v[pl.ds(4 * i, _LANES)]
    tcx = jnp.broadcast_to(bvec[0], (_LANES,)) * 32.0
    tcy = jnp.broadcast_to(bvec[1], (_LANES,)) * 32.0
    tw = jnp.broadcast_to(bvec[2], (_LANES,)) * 32.0
    th = jnp.broadcast_to(bvec[3], (_LANES,)) * 32.0
    ra = tw / 2.0 + 1e-06
    rb = th / 2.0 + 1e-06
    ia = one / (ra * ra)
    ib = one / (rb * rb)
    qxs = []
    for k in range(4):
      dx = tcx - pxs[k]
      qxs.append(dx * dx * ia)
    cyb = tcy - py_half

    def body(j, yf):
      dy = cyb - yf
      t = dy * dy * ib
      acc = jnp.where(qxs[0] + t < 1.0, weights[0], zero)
      for k in range(1, 4):
        acc = acc + jnp.where(qxs[k] + t < 1.0, weights[k], zero)
      out_v[i, pl.ds(j * _LANES, _LANES)] = acc
      return yf + 2.0

    lax.fori_loop(0, _STEPS, body, jnp.zeros((_LANES,), jnp.float32))
    return carry

  lax.fori_loop(0, _BPW, batch_body, 0)
  pltpu.sync_copy(out_v, out_hbm.at[pl.ds(wid * _BPW, _BPW)])


@jax.jit
def _ellipse_mask(boxes_flat):
  mesh = plsc.VectorSubcoreMesh(core_axis_name="c", subcore_axis_name="s")
  f = functools.partial(
      pl.kernel,
      mesh=mesh,
      out_type=jax.ShapeDtypeStruct((_BATCH, _WORDS), jnp.int32),
      scratch_types=[
          pltpu.VMEM((4 * _BPW + _LANES,), jnp.float32),
          pltpu.VMEM((_BPW, _WORDS), jnp.int32),
      ],
  )(_ellipse_body)
  return f(boxes_flat)


def kernel(pred_logits, boxes):
  del pred_logits
  words = _ellipse_mask(boxes.reshape(_BATCH * 4))
  mask_bytes = lax.bitcast_convert_type(words, jnp.uint8).reshape(_BATCH, _NQ)
  return mask_bytes.astype(jnp.bool_)


# speedup vs baseline: 1.0813x; 1.0292x over previous
import functools

import jax
import jax.numpy as jnp
from jax import lax
from jax.experimental import pallas as pl
from jax.experimental.pallas import tpu as pltpu
from jax.experimental.pallas import tpu_sc as plsc

_NUM_CORES = 1
_NUM_SUBCORES = 16
_NUM_WORKERS = _NUM_CORES * _NUM_SUBCORES
_BATCH = 256
_BPW = _BATCH // _NUM_WORKERS
_NQ = 1024
_LANES = 16
_WORDS = _NQ // 4
_STEPS = _WORDS // _LANES


def _ellipse_body(boxes_hbm, out_hbm, box_v, out_v):
  wid = lax.axis_index("s") * _NUM_CORES + lax.axis_index("c")
  pltpu.sync_copy(
      boxes_hbm.at[pl.ds(wid * (4 * _BPW), 4 * _BPW)],
      box_v.at[pl.ds(0, 4 * _BPW)],
  )
  lane = lax.iota(jnp.int32, _LANES)
  lane4 = lane * 4
  pxs = [((lane4 + k) & 31).astype(jnp.float32) for k in range(4)]
  py_half = (lane >> 3).astype(jnp.float32)
  weights = [jnp.int32(1 << (8 * k)) for k in range(4)]
  zero = jnp.zeros((_LANES,), jnp.int32)
  one = jnp.ones((_LANES,), jnp.float32)
  def batch_body(i, carry):
    bvec = box_v[pl.ds(4 * i, _LANES)]
    tcx = jnp.broadcast_to(bvec[0], (_LANES,)) * 32.0
    tcy = jnp.broadcast_to(bvec[1], (_LANES,)) * 32.0
    tw = jnp.broadcast_to(bvec[2], (_LANES,)) * 32.0
    th = jnp.broadcast_to(bvec[3], (_LANES,)) * 32.0
    ra = tw / 2.0 + 1e-06
    rb = th / 2.0 + 1e-06
    ia = one / (ra * ra)
    ib = one / (rb * rb)
    qxs = []
    for k in range(4):
      dx = tcx - pxs[k]
      qxs.append(dx * dx * ia)
    cyb = tcy - py_half

    def body(j, yf):
      dy = cyb - yf
      t = dy * dy * ib
      acc = jnp.where(qxs[0] + t < 1.0, weights[0], zero)
      for k in range(1, 4):
        acc = acc + jnp.where(qxs[k] + t < 1.0, weights[k], zero)
      out_v[i, pl.ds(j * _LANES, _LANES)] = acc
      return yf + 2.0

    lax.fori_loop(0, _STEPS, body, jnp.zeros((_LANES,), jnp.float32))
    return carry

  lax.fori_loop(0, _BPW, batch_body, 0)
  pltpu.sync_copy(out_v, out_hbm.at[pl.ds(wid * _BPW, _BPW)])


@jax.jit
def _ellipse_mask(boxes_flat):
  mesh = plsc.VectorSubcoreMesh(core_axis_name="c", subcore_axis_name="s", num_cores=1)
  f = functools.partial(
      pl.kernel,
      mesh=mesh,
      out_type=jax.ShapeDtypeStruct((_BATCH, _WORDS), jnp.int32),
      scratch_types=[
          pltpu.VMEM((4 * _BPW + _LANES,), jnp.float32),
          pltpu.VMEM((_BPW, _WORDS), jnp.int32),
      ],
  )(_ellipse_body)
  return f(boxes_flat)


def kernel(pred_logits, boxes):
  del pred_logits
  words = _ellipse_mask(boxes.reshape(_BATCH * 4))
  mask_bytes = lax.bitcast_convert_type(words, jnp.uint8).reshape(_BATCH, _NQ)
  return mask_bytes.astype(jnp.bool_)


# speedup vs baseline: 1.0885x; 1.0067x over previous
import functools

import jax
import jax.numpy as jnp
from jax import lax
from jax.experimental import pallas as pl
from jax.experimental.pallas import tpu as pltpu
from jax.experimental.pallas import tpu_sc as plsc

_NUM_CORES = 1
_NUM_SUBCORES = 16
_NUM_WORKERS = _NUM_CORES * _NUM_SUBCORES
_BATCH = 256
_BPW = _BATCH // _NUM_WORKERS
_NQ = 1024
_LANES = 16
_WORDS = _NQ // 4
_STEPS = _WORDS // _LANES


def _ellipse_body(boxes_hbm, out_hbm, box_v, out_v):
  wid = lax.axis_index("s") * _NUM_CORES + lax.axis_index("c")
  pltpu.sync_copy(
      boxes_hbm.at[pl.ds(wid * (4 * _BPW), 4 * _BPW)],
      box_v.at[pl.ds(0, 4 * _BPW)],
  )
  lane = lax.iota(jnp.int32, _LANES)
  lane4 = lane * 4
  pxs = [((lane4 + k) & 31).astype(jnp.float32) for k in range(4)]
  py_half = (lane >> 3).astype(jnp.float32)
  weights = [jnp.int32(1 << (8 * k)) for k in range(4)]
  zero = jnp.zeros((_LANES,), jnp.int32)
  one = jnp.ones((_LANES,), jnp.float32)
  def batch_body(i, carry):
    bvec = box_v[pl.ds(4 * i, _LANES)]
    tcx = jnp.broadcast_to(bvec[0], (_LANES,)) * 32.0
    tcy = jnp.broadcast_to(bvec[1], (_LANES,)) * 32.0
    tw = jnp.broadcast_to(bvec[2], (_LANES,)) * 32.0
    th = jnp.broadcast_to(bvec[3], (_LANES,)) * 32.0
    ra = tw / 2.0 + 1e-06
    rb = th / 2.0 + 1e-06
    ia = one / (ra * ra)
    ib = one / (rb * rb)
    qxs = []
    for k in range(4):
      dx = tcx - pxs[k]
      qxs.append(dx * dx * ia)
    cyb = tcy - py_half

    def body(j, yf):
      for u in range(4):
        dy = cyb - (yf + float(2 * u))
        lim = one - dy * dy * ib
        acc = jnp.where(qxs[0] < lim, weights[0], zero)
        for k in range(1, 4):
          acc = acc + jnp.where(qxs[k] < lim, weights[k], zero)
        out_v[i, pl.ds((j + u) * _LANES, _LANES)] = acc
      return yf + 8.0

    lax.fori_loop(0, _STEPS // 4, lambda jj, yf: body(jj * 4, yf), jnp.zeros((_LANES,), jnp.float32))
    return carry

  lax.fori_loop(0, _BPW, batch_body, 0)
  pltpu.sync_copy(out_v, out_hbm.at[pl.ds(wid * _BPW, _BPW)])


@jax.jit
def _ellipse_mask(boxes_flat):
  mesh = plsc.VectorSubcoreMesh(core_axis_name="c", subcore_axis_name="s", num_cores=1)
  f = functools.partial(
      pl.kernel,
      mesh=mesh,
      out_type=jax.ShapeDtypeStruct((_BATCH, _WORDS), jnp.int32),
      scratch_types=[
          pltpu.VMEM((4 * _BPW + _LANES,), jnp.float32),
          pltpu.VMEM((_BPW, _WORDS), jnp.int32),
      ],
  )(_ellipse_body)
  return f(boxes_flat)


def kernel(pred_logits, boxes):
  del pred_logits
  words = _ellipse_mask(boxes.reshape(_BATCH * 4))
  mask_bytes = lax.bitcast_convert_type(words, jnp.uint8).reshape(_BATCH, _NQ)
  return mask_bytes.astype(jnp.bool_)


# speedup vs baseline: 1.0999x; 1.0104x over previous
import functools

import jax
import jax.numpy as jnp
from jax import lax
from jax.experimental import pallas as pl
from jax.experimental.pallas import tpu as pltpu
from jax.experimental.pallas import tpu_sc as plsc

_NUM_CORES = 1
_NUM_SUBCORES = 16
_NUM_WORKERS = _NUM_CORES * _NUM_SUBCORES
_BATCH = 256
_BPW = _BATCH // _NUM_WORKERS
_NQ = 1024
_LANES = 16
_WORDS = _NQ // 4
_STEPS = _WORDS // _LANES


def _ellipse_body(boxes_hbm, out_hbm, box_v, out_v):
  wid = lax.axis_index("s") * _NUM_CORES + lax.axis_index("c")
  pltpu.sync_copy(
      boxes_hbm.at[pl.ds(wid * (4 * _BPW), 4 * _BPW)],
      box_v.at[pl.ds(0, 4 * _BPW)],
  )
  lane = lax.iota(jnp.int32, _LANES)
  lane4 = lane * 4
  pxs = [((lane4 + k) & 31).astype(jnp.float32) for k in range(4)]
  py_half = (lane >> 3).astype(jnp.float32)
  weights = [jnp.int32(1 << (8 * k)) for k in range(4)]
  zero = jnp.zeros((_LANES,), jnp.int32)
  one = jnp.ones((_LANES,), jnp.float32)
  @plsc.parallel_loop(0, _BPW)
  def batch_body(i):
    bvec = box_v[pl.ds(4 * i, _LANES)]
    tcx = jnp.broadcast_to(bvec[0], (_LANES,)) * 32.0
    tcy = jnp.broadcast_to(bvec[1], (_LANES,)) * 32.0
    tw = jnp.broadcast_to(bvec[2], (_LANES,)) * 32.0
    th = jnp.broadcast_to(bvec[3], (_LANES,)) * 32.0
    ra = tw / 2.0 + 1e-06
    rb = th / 2.0 + 1e-06
    ia = one / (ra * ra)
    ib = one / (rb * rb)
    qxs = []
    for k in range(4):
      dx = tcx - pxs[k]
      qxs.append(dx * dx * ia)
    cyb = tcy - py_half

    @plsc.parallel_loop(0, _STEPS, unroll=4)
    def step_body(j):
      yv = jnp.broadcast_to(j.astype(jnp.float32), (_LANES,)) * 2.0
      dy = cyb - yv
      lim = one - dy * dy * ib
      acc = jnp.where(qxs[0] < lim, weights[0], zero)
      for k in range(1, 4):
        acc = acc + jnp.where(qxs[k] < lim, weights[k], zero)
      out_v[i, pl.ds(j * _LANES, _LANES)] = acc
  pltpu.sync_copy(out_v, out_hbm.at[pl.ds(wid * _BPW, _BPW)])


@jax.jit
def _ellipse_mask(boxes_flat):
  mesh = plsc.VectorSubcoreMesh(core_axis_name="c", subcore_axis_name="s", num_cores=1)
  f = functools.partial(
      pl.kernel,
      mesh=mesh,
      out_type=jax.ShapeDtypeStruct((_BATCH, _WORDS), jnp.int32),
      scratch_types=[
          pltpu.VMEM((4 * _BPW + _LANES,), jnp.float32),
          pltpu.VMEM((_BPW, _WORDS), jnp.int32),
      ],
  )(_ellipse_body)
  return f(boxes_flat)


def kernel(pred_logits, boxes):
  del pred_logits
  words = _ellipse_mask(boxes.reshape(_BATCH * 4))
  mask_bytes = lax.bitcast_convert_type(words, jnp.uint8).reshape(_BATCH, _NQ)
  return mask_bytes.astype(jnp.bool_)
